# baseline (speedup 1.0000x reference)
import functools

import jax
import jax.numpy as jnp
from jax import lax
from jax.experimental import pallas as pl
from jax.experimental.pallas import tpu as pltpu
from jax.experimental.pallas import tpu_sc as plsc

N = 10000
E = 320000
D = 128
DE = 16
H = 128
C16 = 16

NC = 2
NS = 16
NW = NC * NS
EW = E // NW
CH = 80
NCH = EW // CH
RPT = N // NS
RC16 = 125
NRC16 = RPT // RC16
NP = 10240
RPTW = NP // NS
RCW = 128
NRCW = RPTW // RCW

def _silu(x):
    return x * jax.nn.sigmoid(x)


def _pack_pair(a, b):
    au = lax.bitcast_convert_type(a.astype(jnp.bfloat16), jnp.uint16)
    bu = lax.bitcast_convert_type(b.astype(jnp.bfloat16), jnp.uint16)
    packed = au.astype(jnp.uint32) | (bu.astype(jnp.uint32) << 16)
    return lax.bitcast_convert_type(packed, jnp.int32)


def _k1_body(nf, w1d, w1s, be1, td_o, ts_o):
    x = nf[...]
    td = jnp.dot(x, w1d[...], preferred_element_type=jnp.float32) + be1[...]
    ts = jnp.dot(x, w1s[...], preferred_element_type=jnp.float32)
    td_o[...] = td.astype(jnp.bfloat16)
    ts_o[...] = ts.astype(jnp.bfloat16)


def _k1(nf, w1d, w1s, be1):
    B = 1000
    return pl.pallas_call(
        _k1_body,
        grid=(N // B,),
        in_specs=[
            pl.BlockSpec((B, D), lambda i: (i, 0)),
            pl.BlockSpec((D, H), lambda i: (0, 0)),
            pl.BlockSpec((D, H), lambda i: (0, 0)),
            pl.BlockSpec((1, H), lambda i: (0, 0)),
        ],
        out_specs=[
            pl.BlockSpec((B, H), lambda i: (i, 0)),
            pl.BlockSpec((B, H), lambda i: (i, 0)),
        ],
        out_shape=[
            jax.ShapeDtypeStruct((N, H), jnp.bfloat16),
            jax.ShapeDtypeStruct((N, H), jnp.bfloat16),
        ],
    )(nf, w1d, w1s, be1)


NB2 = 4
NT2 = NCH // NB2 - 1


def _gather_ring(wid, specs, gsems, wsems):
    def issue_gathers(b, j):
        return [pltpu.async_copy(t.at[ix.at[j]], bufs[b], gsems[b])
                for (t, ix, bufs, out) in specs]

    def issue_writes(b, j):
        for (t, ix, bufs, out) in specs:
            pltpu.async_copy(bufs[b], out.at[wid, j], wsems[b])

    def drain_writes(b):
        for (t, ix, bufs, out) in specs:
            pltpu.make_async_copy(bufs[b], out.at[wid, 0], wsems[b]).wait()

    cps = [issue_gathers(b, b) for b in range(NB2)]
    for b in range(NB2):
        for cp in cps[b]:
            cp.wait()
        issue_writes(b, b)

    def body_with_drain(t, carry):
        j0 = t * NB2
        for b in range(NB2):
            drain_writes(b)
        cps = [issue_gathers(b, j0 + b) for b in range(NB2)]
        for b in range(NB2):
            for cp in cps[b]:
                cp.wait()
            issue_writes(b, j0 + b)
        return carry

    lax.fori_loop(1, NT2 + 1, body_with_drain, 0)
    for b in range(NB2):
        drain_writes(b)
    for j in range(NB2 * (NT2 + 1), NCH):
        cps = issue_gathers(0, j)
        for cp in cps:
            cp.wait()
        issue_writes(0, j)
        drain_writes(0)


def _k2w_body(td, ts, dst3, src3, gd_o, gs_o,
              idxd, idxs, gds, gss, gsems, wsems):
    c = lax.axis_index("c")
    s = lax.axis_index("s")
    wid = s * NC + c
    pltpu.sync_copy(dst3.at[wid], idxd)
    pltpu.sync_copy(src3.at[wid], idxs)
    _gather_ring(wid, [(td, idxd, gds, gd_o), (ts, idxs, gss, gs_o)],
                 gsems, wsems)


def _k2n_body(cpad, dst3, src3, cd_o, cs_o,
              idxd, idxs, cds, css, gsems, wsems):
    c = lax.axis_index("c")
    s = lax.axis_index("s")
    wid = s * NC + c
    pltpu.sync_copy(dst3.at[wid], idxd)
    pltpu.sync_copy(src3.at[wid], idxs)
    _gather_ring(wid, [(cpad, idxd, cds, cd_o), (cpad, idxs, css, cs_o)],
                 gsems, wsems)


def _unpack_lo(ref, B):
    x = ref[...].reshape(B, H // 2)
    return lax.bitcast_convert_type(jnp.left_shift(x, 16), jnp.float32)


def _unpack_hi(ref, B):
    x = ref[...].reshape(B, H // 2)
    return lax.bitcast_convert_type(
        jnp.bitwise_and(x, jnp.int32(-65536)), jnp.float32)


def _k3_body(gd, gs, cd, cs, ef, w1e, wr, we2, be2, wa, ba, wc1, bc1, wc2,
             u_o, uc_o):
    B = K3B * CH
    diff = cd[...].reshape(B, C16) - cs[...].reshape(B, C16)
    radial = jnp.sum(diff * diff, axis=1, keepdims=True)
    pre = (gd[...].reshape(B, H).astype(jnp.float32)
           + gs[...].reshape(B, H).astype(jnp.float32) + radial * wr[...]
           + jnp.dot(ef[...].reshape(B, DE), w1e[...],
                     preferred_element_type=jnp.float32))
    m = _silu(pre)
    mb = m.astype(jnp.bfloat16)
    m = _silu(jnp.dot(mb, we2[...], preferred_element_type=jnp.float32)
              + be2[...])
    mb = m.astype(jnp.bfloat16)
    att = jax.nn.sigmoid(
        jnp.dot(mb, wa[...], preferred_element_type=jnp.float32) + ba[...])
    m = m * att
    mb = m.astype(jnp.bfloat16)
    cmlp = _silu(jnp.dot(mb, wc1[...], preferred_element_type=jnp.float32)
                 + bc1[...])
    sc = jnp.dot(cmlp.astype(jnp.bfloat16), wc2[...],
                 preferred_element_type=jnp.float32)
    lane = lax.broadcasted_iota(jnp.int32, (1, C16), 1)
    ones_col = (lane == 3).astype(jnp.float32)
    u_o[...] = m.reshape(1, K3B, CH, H)
    uc_o[...] = (diff * sc + ones_col).reshape(1, K3B, CH, C16)


K3B = 25
PK = CH * C16 // 128


def _k3(gd, gs, cd, cs, ef, w1e, wr, we2, be2, wa, ba, wc1, bc1, wc2):
    G = NCH // K3B
    full = lambda i: (0, 0)
    em = lambda i: (i // G, i % G, 0, 0)
    return pl.pallas_call(
        _k3_body,
        grid=(NW * G,),
        in_specs=[
            pl.BlockSpec((1, K3B, CH, H), em),
            pl.BlockSpec((1, K3B, CH, H), em),
            pl.BlockSpec((1, K3B, CH, C16), em),
            pl.BlockSpec((1, K3B, CH, C16), em),
            pl.BlockSpec((1, K3B, CH, DE), em),
            pl.BlockSpec((DE, H), full),
            pl.BlockSpec((1, H), full),
            pl.BlockSpec((H, H), full),
            pl.BlockSpec((1, H), full),
            pl.BlockSpec((H, 1), full),
            pl.BlockSpec((1, 1), full),
            pl.BlockSpec((H, H), full),
            pl.BlockSpec((1, H), full),
            pl.BlockSpec((H, 1), full),
        ],
        out_specs=[
            pl.BlockSpec((1, K3B, CH, H), em),
            pl.BlockSpec((1, K3B, CH, C16), em),
        ],
        out_shape=[
            jax.ShapeDtypeStruct((NW, NCH, CH, H), jnp.float32),
            jax.ShapeDtypeStruct((NW, NCH, CH, C16), jnp.float32),
        ],
    )(gd, gs, cd, cs, ef, w1e, wr, we2, be2, wa, ba, wc1, bc1, wc2)


NB4 = 3
NT4 = NCH // NB4 - 1


def _scatter_ring(wid, dst3, src4, idxbufs, bufs, acc, lsems, ssems):
    def issue_loads(b, j):
        return (pltpu.async_copy(dst3.at[wid, pl.ds(j, 1)], idxbufs[b],
                                 lsems[b]),
                pltpu.async_copy(src4.at[wid, j], bufs[b], lsems[b]))

    def issue_scatters(b):
        pltpu.async_copy(bufs[b], acc.at[idxbufs[b].at[0]], ssems[b],
                         add=True)

    def drain_scatters(b):
        pltpu.make_async_copy(bufs[b], acc.at[pl.ds(0, CH)], ssems[b]).wait()

    cps = [issue_loads(b, b) for b in range(NB4)]
    for b in range(NB4):
        for cp in cps[b]:
            cp.wait()
        issue_scatters(b)

    def body(t, carry):
        j0 = t * NB4
        for b in range(NB4):
            drain_scatters(b)
        cps = [issue_loads(b, j0 + b) for b in range(NB4)]
        for b in range(NB4):
            for cp in cps[b]:
                cp.wait()
            issue_scatters(b)
        return carry

    lax.fori_loop(1, NT4 + 1, body, 0)
    for b in range(NB4):
        drain_scatters(b)
    for j in range(NB4 * (NT4 + 1), NCH):
        cps = issue_loads(0, j)
        for cp in cps:
            cp.wait()
        issue_scatters(0)
        drain_scatters(0)


def _zero_and_run(c, s, z, o, acc, st, rows, nrc, rpt, run):
    r0 = s * rpt
    for k in range(nrc):
        pltpu.sync_copy(z.at[pl.ds(r0 + k * rows, rows)], st)
        pltpu.sync_copy(st, acc.at[pl.ds(r0 + k * rows, rows)])
    plsc.subcore_barrier()
    run()
    plsc.subcore_barrier()
    for k in range(nrc):
        pltpu.sync_copy(acc.at[pl.ds(r0 + k * rows, rows)], st)
        pltpu.sync_copy(st, o.at[c, pl.ds(r0 + k * rows, rows)])


def _k4w_body(u, dst3, z128, o128, idxbufs, ubs, st128, acc128, lsems, ssems):
    c = lax.axis_index("c")
    s = lax.axis_index("s")
    wid = s * NC + c
    _zero_and_run(
        c, s, z128, o128, acc128, st128, RCW, NRCW, RPTW,
        lambda: _scatter_ring(wid, dst3, u, idxbufs, ubs, acc128,
                              lsems, ssems))


def _k4n_body(uc, dst3, z16, o16, idxbufs, ucbs, st16, acc16, lsems, ssems):
    c = lax.axis_index("c")
    s = lax.axis_index("s")
    wid = s * NC + c
    _zero_and_run(
        c, s, z16, o16, acc16, st16, RC16, NRC16, RPT,
        lambda: _scatter_ring(wid, dst3, uc, idxbufs, ucbs, acc16,
                              lsems, ssems))


_sc_cache = {}


def _sc_kernels():
    if "k2w" not in _sc_cache:
        mesh = plsc.VectorSubcoreMesh(
            core_axis_name="c", subcore_axis_name="s",
            num_cores=NC, num_subcores=NS)
        tiled = pltpu.CompilerParams(use_tc_tiling_on_sc=True)
        linear = pltpu.CompilerParams(use_tc_tiling_on_sc=False)
        _sc_cache["k2w"] = pl.kernel(
            _k2w_body,
            out_type=[
                jax.ShapeDtypeStruct((NW, NCH, CH, H), jnp.bfloat16),
                jax.ShapeDtypeStruct((NW, NCH, CH, H), jnp.bfloat16),
            ],
            mesh=mesh,
            scratch_types=[
                pltpu.VMEM((NCH, CH), jnp.int32),
                pltpu.VMEM((NCH, CH), jnp.int32),
                [pltpu.VMEM((CH, H), jnp.bfloat16) for _ in range(NB2)],
                [pltpu.VMEM((CH, H), jnp.bfloat16) for _ in range(NB2)],
                [pltpu.SemaphoreType.DMA for _ in range(NB2)],
                [pltpu.SemaphoreType.DMA for _ in range(NB2)],
            ],
            compiler_params=linear,
        )
        _sc_cache["k2n"] = pl.kernel(
            _k2n_body,
            out_type=[
                jax.ShapeDtypeStruct((NW, NCH, CH, C16), jnp.float32),
                jax.ShapeDtypeStruct((NW, NCH, CH, C16), jnp.float32),
            ],
            mesh=mesh,
            scratch_types=[
                pltpu.VMEM((NCH, CH), jnp.int32),
                pltpu.VMEM((NCH, CH), jnp.int32),
                [pltpu.VMEM((CH, C16), jnp.float32) for _ in range(NB2)],
                [pltpu.VMEM((CH, C16), jnp.float32) for _ in range(NB2)],
                [pltpu.SemaphoreType.DMA for _ in range(NB2)],
                [pltpu.SemaphoreType.DMA for _ in range(NB2)],
            ],
            compiler_params=linear,
        )
        _sc_cache["k4w"] = pl.kernel(
            _k4w_body,
            out_type=[jax.ShapeDtypeStruct((NC, NP, H), jnp.float32)],
            mesh=mesh,
            scratch_types=[
                [pltpu.VMEM((1, CH), jnp.int32) for _ in range(NB4)],
                [pltpu.VMEM((CH, H), jnp.float32) for _ in range(NB4)],
                pltpu.VMEM((RCW, H), jnp.float32),
                pltpu.VMEM_SHARED((NP, H), jnp.float32),
                [pltpu.SemaphoreType.DMA for _ in range(NB4)],
                [pltpu.SemaphoreType.DMA for _ in range(NB4)],
            ],
            compiler_params=tiled,
        )
        _sc_cache["k4n"] = pl.kernel(
            _k4n_body,
            out_type=[jax.ShapeDtypeStruct((NC, N, C16), jnp.float32)],
            mesh=mesh,
            scratch_types=[
                [pltpu.VMEM((1, CH), jnp.int32) for _ in range(NB4)],
                [pltpu.VMEM((CH, C16), jnp.float32) for _ in range(NB4)],
                pltpu.VMEM((RC16, C16), jnp.float32),
                pltpu.VMEM_SHARED((N, C16), jnp.float32),
                [pltpu.SemaphoreType.DMA for _ in range(NB4)],
                [pltpu.SemaphoreType.DMA for _ in range(NB4)],
            ],
            compiler_params=linear,
        )
    return (_sc_cache["k2w"], _sc_cache["k2n"], _sc_cache["k4w"],
            _sc_cache["k4n"])


def _k5_body(nf, cpad, a0, a1, q0, q1, wn1h, wn1a, bn1, wn2, bn2,
             h_o, c_o):
    x = nf[...]
    agg = a0[...] + a1[...]
    h1 = _silu(jnp.dot(x, wn1h[...], preferred_element_type=jnp.float32)
               + jnp.dot(agg, wn1a[...], preferred_element_type=jnp.float32)
               + bn1[...])
    h2 = jnp.dot(h1, wn2[...], preferred_element_type=jnp.float32) + bn2[...]
    h_o[...] = x + h2
    q = q0[...] + q1[...]
    counts = jnp.maximum(q[:, 3:4], 1.0)
    lane = lax.broadcasted_iota(jnp.int32, (1, C16), 1)
    mask3 = (lane < 3).astype(jnp.float32)
    c_o[...] = cpad[...] + (q / counts) * mask3


def _k5(nf, cpad, a0, a1, q0, q1, wn1h, wn1a, bn1, wn2, bn2):
    B = 1000
    full = lambda i: (0, 0)
    return pl.pallas_call(
        _k5_body,
        grid=(N // B,),
        in_specs=[
            pl.BlockSpec((B, D), lambda i: (i, 0)),
            pl.BlockSpec((B, C16), lambda i: (i, 0)),
            pl.BlockSpec((B, H), lambda i: (i, 0)),
            pl.BlockSpec((B, H), lambda i: (i, 0)),
            pl.BlockSpec((B, C16), lambda i: (i, 0)),
            pl.BlockSpec((B, C16), lambda i: (i, 0)),
            pl.BlockSpec((D, H), full),
            pl.BlockSpec((H, H), full),
            pl.BlockSpec((1, H), full),
            pl.BlockSpec((H, D), full),
            pl.BlockSpec((1, D), full),
        ],
        out_specs=[
            pl.BlockSpec((B, D), lambda i: (i, 0)),
            pl.BlockSpec((B, C16), lambda i: (i, 0)),
        ],
        out_shape=[
            jax.ShapeDtypeStruct((N, D), jnp.float32),
            jax.ShapeDtypeStruct((N, C16), jnp.float32),
        ],
    )(nf, cpad, a0, a1, q0, q1, wn1h, wn1a, bn1, wn2, bn2)


def kernel(node_feats, coords, edge_index, edge_feats, W_e1, b_e1, W_e2,
           b_e2, W_n1, b_n1, W_n2, b_n2, W_c1, b_c1, W_c2, W_a, b_a):
    w1d = W_e1[0:D]
    w1s = W_e1[D:2 * D]
    wr = W_e1[2 * D:2 * D + 1]
    w1e = W_e1[2 * D + 1:]
    be1 = b_e1.reshape(1, H)

    td, ts = _k1(node_feats, w1d, w1s, be1)

    cpad = jnp.pad(coords, ((0, 0), (0, C16 - 3)))
    dst3 = edge_index[1].reshape(NW, NCH, CH)
    src3 = edge_index[0].reshape(NW, NCH, CH)

    k2w, k2n, k4w, k4n = _sc_kernels()
    gd, gs = k2w(td, ts, dst3, src3)
    cd, cs = k2n(cpad, dst3, src3)

    bf = jnp.bfloat16
    u, ucol = _k3(
        gd, gs, cd, cs,
        edge_feats.reshape(NW, NCH, CH, DE), w1e, wr, W_e2.astype(bf),
        b_e2.reshape(1, H), W_a.astype(bf), b_a.reshape(1, 1),
        W_c1.astype(bf), b_c1.reshape(1, H), W_c2.astype(bf))

    z128 = jnp.zeros((NP, H), jnp.float32)
    z16 = jnp.zeros((N, C16), jnp.float32)
    (o128,) = k4w(u, dst3, z128)
    (o16,) = k4n(ucol, dst3, z16)

    h_out, c_out = _k5(
        node_feats, cpad, o128[0], o128[1], o16[0], o16[1],
        W_n1[0:D], W_n1[D:], b_n1.reshape(1, H), W_n2, b_n2.reshape(1, D))

    return (h_out, c_out[:, 0:3])


# speedup vs baseline: 1.2700x; 1.2700x over previous
import functools

import jax
import jax.numpy as jnp
from jax import lax
from jax.experimental import pallas as pl
from jax.experimental.pallas import tpu as pltpu
from jax.experimental.pallas import tpu_sc as plsc

N = 10000
E = 320000
D = 128
DE = 16
H = 128
C16 = 16

NC = 2
NS = 16
NW = NC * NS
EW = E // NW
CH = 80
NCH = EW // CH
RPT = N // NS
RC16 = 125
NRC16 = RPT // RC16
NP = 10240
RPTW = NP // NS
RCW = 128
NRCW = RPTW // RCW

def _silu(x):
    return x * jax.nn.sigmoid(x)


def _pack_pair(a, b):
    au = lax.bitcast_convert_type(a.astype(jnp.bfloat16), jnp.uint16)
    bu = lax.bitcast_convert_type(b.astype(jnp.bfloat16), jnp.uint16)
    packed = au.astype(jnp.uint32) | (bu.astype(jnp.uint32) << 16)
    return lax.bitcast_convert_type(packed, jnp.int32)


def _k1_body(nf, w1d, w1s, be1, td_o, ts_o):
    x = nf[...]
    td_o[...] = jnp.dot(x, w1d[...], preferred_element_type=jnp.float32) + be1[...]
    ts_o[...] = jnp.dot(x, w1s[...], preferred_element_type=jnp.float32)


def _k1(nf, w1d, w1s, be1):
    B = 1000
    return pl.pallas_call(
        _k1_body,
        grid=(N // B,),
        in_specs=[
            pl.BlockSpec((B, D), lambda i: (i, 0)),
            pl.BlockSpec((D, H), lambda i: (0, 0)),
            pl.BlockSpec((D, H), lambda i: (0, 0)),
            pl.BlockSpec((1, H), lambda i: (0, 0)),
        ],
        out_specs=[
            pl.BlockSpec((B, H), lambda i: (i, 0)),
            pl.BlockSpec((B, H), lambda i: (i, 0)),
        ],
        out_shape=[
            jax.ShapeDtypeStruct((N, H), jnp.float32),
            jax.ShapeDtypeStruct((N, H), jnp.float32),
        ],
    )(nf, w1d, w1s, be1)


NB2 = 4
NT2 = NCH // NB2 - 1


def _gather_ring(wid, specs, gsems, wsems, nch):
    nt = nch // NB2 - 1

    def issue_gathers(b, j):
        return [pltpu.async_copy(t.at[ix.at[j]], bufs[b], gsems[b])
                for (t, ix, bufs, out) in specs]

    def issue_writes(b, j):
        for (t, ix, bufs, out) in specs:
            pltpu.async_copy(bufs[b], out.at[wid, j], wsems[b])

    def drain_writes(b):
        for (t, ix, bufs, out) in specs:
            pltpu.make_async_copy(bufs[b], out.at[wid, 0], wsems[b]).wait()

    cps = [issue_gathers(b, b) for b in range(NB2)]
    for b in range(NB2):
        for cp in cps[b]:
            cp.wait()
        issue_writes(b, b)

    def body_with_drain(t, carry):
        j0 = t * NB2
        for b in range(NB2):
            drain_writes(b)
        cps = [issue_gathers(b, j0 + b) for b in range(NB2)]
        for b in range(NB2):
            for cp in cps[b]:
                cp.wait()
            issue_writes(b, j0 + b)
        return carry

    lax.fori_loop(1, nt + 1, body_with_drain, 0)
    for b in range(NB2):
        drain_writes(b)
    for j in range(NB2 * (nt + 1), nch):
        cps = issue_gathers(0, j)
        for cp in cps:
            cp.wait()
        issue_writes(0, j)
        drain_writes(0)


def _k2w_body(td, ts, dst3, src3, gd_o, gs_o,
              idxd, idxs, gds, gss, gsems, wsems, nch):
    c = lax.axis_index("c")
    s = lax.axis_index("s")
    wid = s * NC + c
    pltpu.sync_copy(dst3.at[wid], idxd)
    pltpu.sync_copy(src3.at[wid], idxs)
    _gather_ring(wid, [(td, idxd, gds, gd_o), (ts, idxs, gss, gs_o)],
                 gsems, wsems, nch)


def _k2n_body(cpad, dst3, src3, cd_o, cs_o,
              idxd, idxs, cds, css, gsems, wsems, nch):
    c = lax.axis_index("c")
    s = lax.axis_index("s")
    wid = s * NC + c
    pltpu.sync_copy(dst3.at[wid], idxd)
    pltpu.sync_copy(src3.at[wid], idxs)
    _gather_ring(wid, [(cpad, idxd, cds, cd_o), (cpad, idxs, css, cs_o)],
                 gsems, wsems, nch)


def _unpack_lo(ref, B):
    x = ref[...].reshape(B, H // 2)
    return lax.bitcast_convert_type(jnp.left_shift(x, 16), jnp.float32)


def _unpack_hi(ref, B):
    x = ref[...].reshape(B, H // 2)
    return lax.bitcast_convert_type(
        jnp.bitwise_and(x, jnp.int32(-65536)), jnp.float32)


def _k3_body(gd, gs, cd, cs, ef, w1e, wr, we2, be2, wa, ba, wc1, bc1, wc2,
             u_o, uc_o, k3b):
    B = k3b * CH
    diff = cd[...].reshape(B, C16) - cs[...].reshape(B, C16)
    radial = jnp.sum(diff * diff, axis=1, keepdims=True)
    pre = (gd[...].reshape(B, H).astype(jnp.float32)
           + gs[...].reshape(B, H).astype(jnp.float32) + radial * wr[...]
           + jnp.dot(ef[...].reshape(B, DE), w1e[...],
                     preferred_element_type=jnp.float32))
    m = _silu(pre)
    m = _silu(jnp.dot(m, we2[...], preferred_element_type=jnp.float32)
              + be2[...])
    att = jax.nn.sigmoid(
        jnp.dot(m, wa[...], preferred_element_type=jnp.float32) + ba[...])
    m = m * att
    cmlp = _silu(jnp.dot(m, wc1[...], preferred_element_type=jnp.float32)
                 + bc1[...])
    sc = jnp.dot(cmlp, wc2[...],
                 preferred_element_type=jnp.float32)
    lane = lax.broadcasted_iota(jnp.int32, (1, C16), 1)
    ones_col = (lane == 3).astype(jnp.float32)
    u_o[...] = m.reshape(1, k3b, CH, H)
    uc_o[...] = (diff * sc + ones_col).reshape(1, k3b, CH, C16)


K3B = 25
NCHA = 60
K3BA = 20
K3BB = 13
PK = CH * C16 // 128


def _k3(gd, gs, cd, cs, ef, w1e, wr, we2, be2, wa, ba, wc1, bc1, wc2,
        nch, k3b):
    G = nch // k3b
    full = lambda i: (0, 0)
    em = lambda i: (i // G, i % G, 0, 0)
    return pl.pallas_call(
        functools.partial(_k3_body, k3b=k3b),
        grid=(NW * G,),
        in_specs=[
            pl.BlockSpec((1, k3b, CH, H), em),
            pl.BlockSpec((1, k3b, CH, H), em),
            pl.BlockSpec((1, k3b, CH, C16), em),
            pl.BlockSpec((1, k3b, CH, C16), em),
            pl.BlockSpec((1, k3b, CH, DE), em),
            pl.BlockSpec((DE, H), full),
            pl.BlockSpec((1, H), full),
            pl.BlockSpec((H, H), full),
            pl.BlockSpec((1, H), full),
            pl.BlockSpec((H, 1), full),
            pl.BlockSpec((1, 1), full),
            pl.BlockSpec((H, H), full),
            pl.BlockSpec((1, H), full),
            pl.BlockSpec((H, 1), full),
        ],
        out_specs=[
            pl.BlockSpec((1, k3b, CH, H), em),
            pl.BlockSpec((1, k3b, CH, C16), em),
        ],
        out_shape=[
            jax.ShapeDtypeStruct((NW, nch, CH, H), jnp.float32),
            jax.ShapeDtypeStruct((NW, nch, CH, C16), jnp.float32),
        ],
    )(gd, gs, cd, cs, ef, w1e, wr, we2, be2, wa, ba, wc1, bc1, wc2)


NB4 = 3
NT4 = NCH // NB4 - 1


def _scatter_ring(wid, dst3, src4, idxbufs, bufs, acc, lsems, ssems, nch):
    nt = nch // NB4 - 1

    def issue_loads(b, j):
        return (pltpu.async_copy(dst3.at[wid, pl.ds(j, 1)], idxbufs[b],
                                 lsems[b]),
                pltpu.async_copy(src4.at[wid, j], bufs[b], lsems[b]))

    def issue_scatters(b):
        pltpu.async_copy(bufs[b], acc.at[idxbufs[b].at[0]], ssems[b],
                         add=True)

    def drain_scatters(b):
        pltpu.make_async_copy(bufs[b], acc.at[pl.ds(0, CH)], ssems[b]).wait()

    cps = [issue_loads(b, b) for b in range(NB4)]
    for b in range(NB4):
        for cp in cps[b]:
            cp.wait()
        issue_scatters(b)

    def body(t, carry):
        j0 = t * NB4
        for b in range(NB4):
            drain_scatters(b)
        cps = [issue_loads(b, j0 + b) for b in range(NB4)]
        for b in range(NB4):
            for cp in cps[b]:
                cp.wait()
            issue_scatters(b)
        return carry

    lax.fori_loop(1, nt + 1, body, 0)
    for b in range(NB4):
        drain_scatters(b)
    for j in range(NB4 * (nt + 1), nch):
        cps = issue_loads(0, j)
        for cp in cps:
            cp.wait()
        issue_scatters(0)
        drain_scatters(0)


def _zero_and_run(c, s, z, o, acc, st, rows, nrc, rpt, run):
    r0 = s * rpt
    for k in range(nrc):
        pltpu.sync_copy(z.at[pl.ds(r0 + k * rows, rows)], st)
        pltpu.sync_copy(st, acc.at[pl.ds(r0 + k * rows, rows)])
    plsc.subcore_barrier()
    run()
    plsc.subcore_barrier()
    for k in range(nrc):
        pltpu.sync_copy(acc.at[pl.ds(r0 + k * rows, rows)], st)
        pltpu.sync_copy(st, o.at[c, pl.ds(r0 + k * rows, rows)])


def _k4w_body(u, dst3, z128, o128, idxbufs, ubs, st128, acc128, lsems,
              ssems, nch):
    c = lax.axis_index("c")
    s = lax.axis_index("s")
    wid = s * NC + c
    _zero_and_run(
        c, s, z128, o128, acc128, st128, RCW, NRCW, RPTW,
        lambda: _scatter_ring(wid, dst3, u, idxbufs, ubs, acc128,
                              lsems, ssems, nch))


def _k4n_body(uc, dst3, z16, o16, idxbufs, ucbs, st16, acc16, lsems,
              ssems, nch):
    c = lax.axis_index("c")
    s = lax.axis_index("s")
    wid = s * NC + c
    _zero_and_run(
        c, s, z16, o16, acc16, st16, RC16, NRC16, RPT,
        lambda: _scatter_ring(wid, dst3, uc, idxbufs, ucbs, acc16,
                              lsems, ssems, nch))


_sc_cache = {}


def _sc_kernels(nch):
    if ("k2w", nch) not in _sc_cache:
        mesh = plsc.VectorSubcoreMesh(
            core_axis_name="c", subcore_axis_name="s",
            num_cores=NC, num_subcores=NS)
        tiled = pltpu.CompilerParams(use_tc_tiling_on_sc=True)
        linear = pltpu.CompilerParams(use_tc_tiling_on_sc=False)
        _sc_cache["k2w", nch] = pl.kernel(
            functools.partial(_k2w_body, nch=nch),
            out_type=[
                jax.ShapeDtypeStruct((NW, nch, CH, H), jnp.float32),
                jax.ShapeDtypeStruct((NW, nch, CH, H), jnp.float32),
            ],
            mesh=mesh,
            scratch_types=[
                pltpu.VMEM((nch, CH), jnp.int32),
                pltpu.VMEM((nch, CH), jnp.int32),
                [pltpu.VMEM((CH, H), jnp.float32) for _ in range(NB2)],
                [pltpu.VMEM((CH, H), jnp.float32) for _ in range(NB2)],
                [pltpu.SemaphoreType.DMA for _ in range(NB2)],
                [pltpu.SemaphoreType.DMA for _ in range(NB2)],
            ],
            compiler_params=tiled,
        )
        _sc_cache["k2n", nch] = pl.kernel(
            functools.partial(_k2n_body, nch=nch),
            out_type=[
                jax.ShapeDtypeStruct((NW, nch, CH, C16), jnp.float32),
                jax.ShapeDtypeStruct((NW, nch, CH, C16), jnp.float32),
            ],
            mesh=mesh,
            scratch_types=[
                pltpu.VMEM((nch, CH), jnp.int32),
                pltpu.VMEM((nch, CH), jnp.int32),
                [pltpu.VMEM((CH, C16), jnp.float32) for _ in range(NB2)],
                [pltpu.VMEM((CH, C16), jnp.float32) for _ in range(NB2)],
                [pltpu.SemaphoreType.DMA for _ in range(NB2)],
                [pltpu.SemaphoreType.DMA for _ in range(NB2)],
            ],
            compiler_params=linear,
        )
        _sc_cache["k4w", nch] = pl.kernel(
            functools.partial(_k4w_body, nch=nch),
            out_type=[jax.ShapeDtypeStruct((NC, NP, H), jnp.float32)],
            mesh=mesh,
            scratch_types=[
                [pltpu.VMEM((1, CH), jnp.int32) for _ in range(NB4)],
                [pltpu.VMEM((CH, H), jnp.float32) for _ in range(NB4)],
                pltpu.VMEM((RCW, H), jnp.float32),
                pltpu.VMEM_SHARED((NP, H), jnp.float32),
                [pltpu.SemaphoreType.DMA for _ in range(NB4)],
                [pltpu.SemaphoreType.DMA for _ in range(NB4)],
            ],
            compiler_params=tiled,
        )
        _sc_cache["k4n", nch] = pl.kernel(
            functools.partial(_k4n_body, nch=nch),
            out_type=[jax.ShapeDtypeStruct((NC, N, C16), jnp.float32)],
            mesh=mesh,
            scratch_types=[
                [pltpu.VMEM((1, CH), jnp.int32) for _ in range(NB4)],
                [pltpu.VMEM((CH, C16), jnp.float32) for _ in range(NB4)],
                pltpu.VMEM((RC16, C16), jnp.float32),
                pltpu.VMEM_SHARED((N, C16), jnp.float32),
                [pltpu.SemaphoreType.DMA for _ in range(NB4)],
                [pltpu.SemaphoreType.DMA for _ in range(NB4)],
            ],
            compiler_params=linear,
        )
    return (_sc_cache["k2w", nch], _sc_cache["k2n", nch],
            _sc_cache["k4w", nch], _sc_cache["k4n", nch])


def _k5_body(nf, cpad, a0, a1, a2, a3, q0, q1, q2, q3, wn1h, wn1a, bn1,
             wn2, bn2, h_o, c_o):
    x = nf[...]
    agg = a0[...] + a1[...] + a2[...] + a3[...]
    h1 = _silu(jnp.dot(x, wn1h[...], preferred_element_type=jnp.float32)
               + jnp.dot(agg, wn1a[...], preferred_element_type=jnp.float32)
               + bn1[...])
    h2 = jnp.dot(h1, wn2[...], preferred_element_type=jnp.float32) + bn2[...]
    h_o[...] = x + h2
    q = q0[...] + q1[...] + q2[...] + q3[...]
    counts = jnp.maximum(q[:, 3:4], 1.0)
    lane = lax.broadcasted_iota(jnp.int32, (1, C16), 1)
    mask3 = (lane < 3).astype(jnp.float32)
    c_o[...] = cpad[...] + (q / counts) * mask3


def _k5(nf, cpad, a0, a1, a2, a3, q0, q1, q2, q3, wn1h, wn1a, bn1, wn2,
        bn2):
    B = 1000
    full = lambda i: (0, 0)
    return pl.pallas_call(
        _k5_body,
        grid=(N // B,),
        in_specs=[
            pl.BlockSpec((B, D), lambda i: (i, 0)),
            pl.BlockSpec((B, C16), lambda i: (i, 0)),
            pl.BlockSpec((B, H), lambda i: (i, 0)),
            pl.BlockSpec((B, H), lambda i: (i, 0)),
            pl.BlockSpec((B, H), lambda i: (i, 0)),
            pl.BlockSpec((B, H), lambda i: (i, 0)),
            pl.BlockSpec((B, C16), lambda i: (i, 0)),
            pl.BlockSpec((B, C16), lambda i: (i, 0)),
            pl.BlockSpec((B, C16), lambda i: (i, 0)),
            pl.BlockSpec((B, C16), lambda i: (i, 0)),
            pl.BlockSpec((D, H), full),
            pl.BlockSpec((H, H), full),
            pl.BlockSpec((1, H), full),
            pl.BlockSpec((H, D), full),
            pl.BlockSpec((1, D), full),
        ],
        out_specs=[
            pl.BlockSpec((B, D), lambda i: (i, 0)),
            pl.BlockSpec((B, C16), lambda i: (i, 0)),
        ],
        out_shape=[
            jax.ShapeDtypeStruct((N, D), jnp.float32),
            jax.ShapeDtypeStruct((N, C16), jnp.float32),
        ],
    )(nf, cpad, a0, a1, a2, a3, q0, q1, q2, q3, wn1h, wn1a, bn1, wn2,
      bn2)


def kernel(node_feats, coords, edge_index, edge_feats, W_e1, b_e1, W_e2,
           b_e2, W_n1, b_n1, W_n2, b_n2, W_c1, b_c1, W_c2, W_a, b_a):
    w1d = W_e1[0:D]
    w1s = W_e1[D:2 * D]
    wr = W_e1[2 * D:2 * D + 1]
    w1e = W_e1[2 * D + 1:]
    be1 = b_e1.reshape(1, H)

    td, ts = _k1(node_feats, w1d, w1s, be1)

    cpad = jnp.pad(coords, ((0, 0), (0, C16 - 3)))
    dst3 = edge_index[1].reshape(NW, NCH, CH)
    src3 = edge_index[0].reshape(NW, NCH, CH)

    z128 = jnp.zeros((NP, H), jnp.float32)
    z16 = jnp.zeros((N, C16), jnp.float32)
    ef4 = edge_feats.reshape(NW, NCH, CH, DE)

    halves = [(0, NCHA, K3BA), (NCHA, NCH, K3BB)]
    o128s, o16s = [], []
    for (j0, j1, k3b) in halves:
        nch = j1 - j0
        k2w, k2n, k4w, k4n = _sc_kernels(nch)
        d3 = dst3[:, j0:j1]
        s3 = src3[:, j0:j1]
        gd, gs = k2w(td, ts, d3, s3)
        cd, cs = k2n(cpad, d3, s3)
        u, ucol = _k3(
            gd, gs, cd, cs, ef4[:, j0:j1], w1e, wr, W_e2,
            b_e2.reshape(1, H), W_a, b_a.reshape(1, 1), W_c1,
            b_c1.reshape(1, H), W_c2, nch, k3b)
        (o128,) = k4w(u, d3, z128)
        (o16,) = k4n(ucol, d3, z16)
        o128s.append(o128)
        o16s.append(o16)

    h_out, c_out = _k5(
        node_feats, cpad, o128s[0][0], o128s[0][1], o128s[1][0],
        o128s[1][1], o16s[0][0], o16s[0][1], o16s[1][0], o16s[1][1],
        W_n1[0:D], W_n1[D:], b_n1.reshape(1, H), W_n2, b_n2.reshape(1, D))

    return (h_out, c_out[:, 0:3])


# speedup vs baseline: 1.4986x; 1.1800x over previous
import functools

import jax
import jax.numpy as jnp
from jax import lax
from jax.experimental import pallas as pl
from jax.experimental.pallas import tpu as pltpu
from jax.experimental.pallas import tpu_sc as plsc

N = 10000
E = 320000
D = 128
DE = 16
H = 128
C16 = 16

NC = 2
NS = 16
NW = NC * NS
EW = E // NW
CH = 80
NCH = EW // CH
RPT = N // NS
RC16 = 125
NRC16 = RPT // RC16
NP = 10240
RPTW = NP // NS
RCW = 128
NRCW = RPTW // RCW

def _silu(x):
    return x * jax.nn.sigmoid(x)


def _pack_pair(a, b):
    au = lax.bitcast_convert_type(a.astype(jnp.bfloat16), jnp.uint16)
    bu = lax.bitcast_convert_type(b.astype(jnp.bfloat16), jnp.uint16)
    packed = au.astype(jnp.uint32) | (bu.astype(jnp.uint32) << 16)
    return lax.bitcast_convert_type(packed, jnp.int32)


def _k1_body(nf, w1d, w1s, be1, td_o, ts_o):
    x = nf[...]
    td_o[...] = jnp.dot(x, w1d[...], preferred_element_type=jnp.float32) + be1[...]
    ts_o[...] = jnp.dot(x, w1s[...], preferred_element_type=jnp.float32)


def _k1(nf, w1d, w1s, be1):
    B = 1000
    return pl.pallas_call(
        _k1_body,
        grid=(N // B,),
        in_specs=[
            pl.BlockSpec((B, D), lambda i: (i, 0)),
            pl.BlockSpec((D, H), lambda i: (0, 0)),
            pl.BlockSpec((D, H), lambda i: (0, 0)),
            pl.BlockSpec((1, H), lambda i: (0, 0)),
        ],
        out_specs=[
            pl.BlockSpec((B, H), lambda i: (i, 0)),
            pl.BlockSpec((B, H), lambda i: (i, 0)),
        ],
        out_shape=[
            jax.ShapeDtypeStruct((N, H), jnp.float32),
            jax.ShapeDtypeStruct((N, H), jnp.float32),
        ],
    )(nf, w1d, w1s, be1)


NB2 = 4
NT2 = NCH // NB2 - 1


def _gather_ring(wid, specs, gsems, wsems):
    def issue_gathers(b, j):
        return [pltpu.async_copy(t.at[ix.at[j]], bufs[b], gsems[b])
                for (t, ix, bufs, out) in specs]

    def issue_writes(b, j):
        for (t, ix, bufs, out) in specs:
            pltpu.async_copy(bufs[b], out.at[wid, j], wsems[b])

    def drain_writes(b):
        for (t, ix, bufs, out) in specs:
            pltpu.make_async_copy(bufs[b], out.at[wid, 0], wsems[b]).wait()

    cps = [issue_gathers(b, b) for b in range(NB2)]
    for b in range(NB2):
        for cp in cps[b]:
            cp.wait()
        issue_writes(b, b)

    def body_with_drain(t, carry):
        j0 = t * NB2
        for b in range(NB2):
            drain_writes(b)
        cps = [issue_gathers(b, j0 + b) for b in range(NB2)]
        for b in range(NB2):
            for cp in cps[b]:
                cp.wait()
            issue_writes(b, j0 + b)
        return carry

    lax.fori_loop(1, NT2 + 1, body_with_drain, 0)
    for b in range(NB2):
        drain_writes(b)
    for j in range(NB2 * (NT2 + 1), NCH):
        cps = issue_gathers(0, j)
        for cp in cps:
            cp.wait()
        issue_writes(0, j)
        drain_writes(0)


def _k2w_body(td, ts, dst3, src3, gd_o, gs_o,
              idxd, idxs, gds, gss, gsems, wsems):
    c = lax.axis_index("c")
    s = lax.axis_index("s")
    wid = s * NC + c
    pltpu.sync_copy(dst3.at[wid], idxd)
    pltpu.sync_copy(src3.at[wid], idxs)
    _gather_ring(wid, [(td, idxd, gds, gd_o), (ts, idxs, gss, gs_o)],
                 gsems, wsems)


def _k2n_body(cpad, dst3, src3, cd_o, cs_o,
              idxd, idxs, cds, css, gsems, wsems):
    c = lax.axis_index("c")
    s = lax.axis_index("s")
    wid = s * NC + c
    pltpu.sync_copy(dst3.at[wid], idxd)
    pltpu.sync_copy(src3.at[wid], idxs)
    _gather_ring(wid, [(cpad, idxd, cds, cd_o), (cpad, idxs, css, cs_o)],
                 gsems, wsems)


def _unpack_lo(ref, B):
    x = ref[...].reshape(B, H // 2)
    return lax.bitcast_convert_type(jnp.left_shift(x, 16), jnp.float32)


def _unpack_hi(ref, B):
    x = ref[...].reshape(B, H // 2)
    return lax.bitcast_convert_type(
        jnp.bitwise_and(x, jnp.int32(-65536)), jnp.float32)


def _k3_body(gd, gs, cd, cs, ef, w1e, wr, we2, be2, wa, ba, wc1, bc1, wc2,
             u_o, uc_o):
    B = K3B * CH
    diff = cd[...].reshape(B, C16) - cs[...].reshape(B, C16)
    radial = jnp.sum(diff * diff, axis=1, keepdims=True)
    pre = (gd[...].reshape(B, H).astype(jnp.float32)
           + gs[...].reshape(B, H).astype(jnp.float32) + radial * wr[...]
           + jnp.dot(ef[...].reshape(B, DE), w1e[...],
                     preferred_element_type=jnp.float32))
    m = _silu(pre)
    m = _silu(jnp.dot(m, we2[...], preferred_element_type=jnp.float32)
              + be2[...])
    att = jax.nn.sigmoid(
        jnp.dot(m, wa[...], preferred_element_type=jnp.float32) + ba[...])
    m = m * att
    cmlp = _silu(jnp.dot(m, wc1[...], preferred_element_type=jnp.float32)
                 + bc1[...])
    sc = jnp.dot(cmlp, wc2[...],
                 preferred_element_type=jnp.float32)
    lane = lax.broadcasted_iota(jnp.int32, (1, C16), 1)
    ones_col = (lane == 3).astype(jnp.float32)
    u_o[...] = m.reshape(1, K3B, CH, H)
    uc_o[...] = (diff * sc + ones_col).reshape(1, K3B, CH, C16)


K3B = 25
PK = CH * C16 // 128


def _k3(gd, gs, cd, cs, ef, w1e, wr, we2, be2, wa, ba, wc1, bc1, wc2):
    G = NCH // K3B
    full = lambda i: (0, 0)
    em = lambda i: (i // G, i % G, 0, 0)
    return pl.pallas_call(
        _k3_body,
        grid=(NW * G,),
        in_specs=[
            pl.BlockSpec((1, K3B, CH, H), em),
            pl.BlockSpec((1, K3B, CH, H), em),
            pl.BlockSpec((1, K3B, CH, C16), em),
            pl.BlockSpec((1, K3B, CH, C16), em),
            pl.BlockSpec((1, K3B, CH, DE), em),
            pl.BlockSpec((DE, H), full),
            pl.BlockSpec((1, H), full),
            pl.BlockSpec((H, H), full),
            pl.BlockSpec((1, H), full),
            pl.BlockSpec((H, 1), full),
            pl.BlockSpec((1, 1), full),
            pl.BlockSpec((H, H), full),
            pl.BlockSpec((1, H), full),
            pl.BlockSpec((H, 1), full),
        ],
        out_specs=[
            pl.BlockSpec((1, K3B, CH, H), em),
            pl.BlockSpec((1, K3B, CH, C16), em),
        ],
        out_shape=[
            jax.ShapeDtypeStruct((NW, NCH, CH, H), jnp.float32),
            jax.ShapeDtypeStruct((NW, NCH, CH, C16), jnp.float32),
        ],
    )(gd, gs, cd, cs, ef, w1e, wr, we2, be2, wa, ba, wc1, bc1, wc2)


NB4 = 3
NT4 = NCH // NB4 - 1


def _scatter_ring(wid, dst3, src4, idxbufs, bufs, acc, lsems, ssems):
    def issue_loads(b, j):
        return (pltpu.async_copy(dst3.at[wid, pl.ds(j, 1)], idxbufs[b],
                                 lsems[b]),
                pltpu.async_copy(src4.at[wid, j], bufs[b], lsems[b]))

    def issue_scatters(b):
        pltpu.async_copy(bufs[b], acc.at[idxbufs[b].at[0]], ssems[b],
                         add=True)

    def drain_scatters(b):
        pltpu.make_async_copy(bufs[b], acc.at[pl.ds(0, CH)], ssems[b]).wait()

    cps = [issue_loads(b, b) for b in range(NB4)]
    for b in range(NB4):
        for cp in cps[b]:
            cp.wait()
        issue_scatters(b)

    def body(t, carry):
        j0 = t * NB4
        for b in range(NB4):
            drain_scatters(b)
        cps = [issue_loads(b, j0 + b) for b in range(NB4)]
        for b in range(NB4):
            for cp in cps[b]:
                cp.wait()
            issue_scatters(b)
        return carry

    lax.fori_loop(1, NT4 + 1, body, 0)
    for b in range(NB4):
        drain_scatters(b)
    for j in range(NB4 * (NT4 + 1), NCH):
        cps = issue_loads(0, j)
        for cp in cps:
            cp.wait()
        issue_scatters(0)
        drain_scatters(0)


def _zero_and_run(c, s, z, o, acc, st, rows, nrc, rpt, run):
    r0 = s * rpt
    for k in range(nrc):
        pltpu.sync_copy(z.at[pl.ds(r0 + k * rows, rows)], st)
        pltpu.sync_copy(st, acc.at[pl.ds(r0 + k * rows, rows)])
    plsc.subcore_barrier()
    run()
    plsc.subcore_barrier()
    for k in range(nrc):
        pltpu.sync_copy(acc.at[pl.ds(r0 + k * rows, rows)], st)
        pltpu.sync_copy(st, o.at[c, pl.ds(r0 + k * rows, rows)])


def _k4w_body(u, dst3, z128, o128, idxbufs, ubs, st128, acc128, lsems, ssems):
    c = lax.axis_index("c")
    s = lax.axis_index("s")
    wid = s * NC + c
    _zero_and_run(
        c, s, z128, o128, acc128, st128, RCW, NRCW, RPTW,
        lambda: _scatter_ring(wid, dst3, u, idxbufs, ubs, acc128,
                              lsems, ssems))


def _k4n_body(uc, dst3, z16, o16, idxbufs, ucbs, st16, acc16, lsems, ssems):
    c = lax.axis_index("c")
    s = lax.axis_index("s")
    wid = s * NC + c
    _zero_and_run(
        c, s, z16, o16, acc16, st16, RC16, NRC16, RPT,
        lambda: _scatter_ring(wid, dst3, uc, idxbufs, ucbs, acc16,
                              lsems, ssems))


_sc_cache = {}


def _sc_kernels():
    if "k2w" not in _sc_cache:
        mesh = plsc.VectorSubcoreMesh(
            core_axis_name="c", subcore_axis_name="s",
            num_cores=NC, num_subcores=NS)
        tiled = pltpu.CompilerParams(use_tc_tiling_on_sc=True)
        linear = pltpu.CompilerParams(use_tc_tiling_on_sc=False)
        _sc_cache["k2w"] = pl.kernel(
            _k2w_body,
            out_type=[
                jax.ShapeDtypeStruct((NW, NCH, CH, H), jnp.float32),
                jax.ShapeDtypeStruct((NW, NCH, CH, H), jnp.float32),
            ],
            mesh=mesh,
            scratch_types=[
                pltpu.VMEM((NCH, CH), jnp.int32),
                pltpu.VMEM((NCH, CH), jnp.int32),
                [pltpu.VMEM((CH, H), jnp.float32) for _ in range(NB2)],
                [pltpu.VMEM((CH, H), jnp.float32) for _ in range(NB2)],
                [pltpu.SemaphoreType.DMA for _ in range(NB2)],
                [pltpu.SemaphoreType.DMA for _ in range(NB2)],
            ],
            compiler_params=tiled,
        )
        _sc_cache["k2n"] = pl.kernel(
            _k2n_body,
            out_type=[
                jax.ShapeDtypeStruct((NW, NCH, CH, C16), jnp.float32),
                jax.ShapeDtypeStruct((NW, NCH, CH, C16), jnp.float32),
            ],
            mesh=mesh,
            scratch_types=[
                pltpu.VMEM((NCH, CH), jnp.int32),
                pltpu.VMEM((NCH, CH), jnp.int32),
                [pltpu.VMEM((CH, C16), jnp.float32) for _ in range(NB2)],
                [pltpu.VMEM((CH, C16), jnp.float32) for _ in range(NB2)],
                [pltpu.SemaphoreType.DMA for _ in range(NB2)],
                [pltpu.SemaphoreType.DMA for _ in range(NB2)],
            ],
            compiler_params=linear,
        )
        _sc_cache["k4w"] = pl.kernel(
            _k4w_body,
            out_type=[jax.ShapeDtypeStruct((NC, NP, H), jnp.float32)],
            mesh=mesh,
            scratch_types=[
                [pltpu.VMEM((1, CH), jnp.int32) for _ in range(NB4)],
                [pltpu.VMEM((CH, H), jnp.float32) for _ in range(NB4)],
                pltpu.VMEM((RCW, H), jnp.float32),
                pltpu.VMEM_SHARED((NP, H), jnp.float32),
                [pltpu.SemaphoreType.DMA for _ in range(NB4)],
                [pltpu.SemaphoreType.DMA for _ in range(NB4)],
            ],
            compiler_params=tiled,
        )
        _sc_cache["k4n"] = pl.kernel(
            _k4n_body,
            out_type=[jax.ShapeDtypeStruct((NC, N, C16), jnp.float32)],
            mesh=mesh,
            scratch_types=[
                [pltpu.VMEM((1, CH), jnp.int32) for _ in range(NB4)],
                [pltpu.VMEM((CH, C16), jnp.float32) for _ in range(NB4)],
                pltpu.VMEM((RC16, C16), jnp.float32),
                pltpu.VMEM_SHARED((N, C16), jnp.float32),
                [pltpu.SemaphoreType.DMA for _ in range(NB4)],
                [pltpu.SemaphoreType.DMA for _ in range(NB4)],
            ],
            compiler_params=linear,
        )
    return (_sc_cache["k2w"], _sc_cache["k2n"], _sc_cache["k4w"],
            _sc_cache["k4n"])


def _k5_body(nf, cpad, a0, a1, q0, q1, wn1h, wn1a, bn1, wn2, bn2,
             h_o, c_o):
    x = nf[...]
    agg = a0[...] + a1[...]
    h1 = _silu(jnp.dot(x, wn1h[...], preferred_element_type=jnp.float32)
               + jnp.dot(agg, wn1a[...], preferred_element_type=jnp.float32)
               + bn1[...])
    h2 = jnp.dot(h1, wn2[...], preferred_element_type=jnp.float32) + bn2[...]
    h_o[...] = x + h2
    q = q0[...] + q1[...]
    counts = jnp.maximum(q[:, 3:4], 1.0)
    lane = lax.broadcasted_iota(jnp.int32, (1, C16), 1)
    mask3 = (lane < 3).astype(jnp.float32)
    c_o[...] = cpad[...] + (q / counts) * mask3


def _k5(nf, cpad, a0, a1, q0, q1, wn1h, wn1a, bn1, wn2, bn2):
    B = 1000
    full = lambda i: (0, 0)
    return pl.pallas_call(
        _k5_body,
        grid=(N // B,),
        in_specs=[
            pl.BlockSpec((B, D), lambda i: (i, 0)),
            pl.BlockSpec((B, C16), lambda i: (i, 0)),
            pl.BlockSpec((B, H), lambda i: (i, 0)),
            pl.BlockSpec((B, H), lambda i: (i, 0)),
            pl.BlockSpec((B, C16), lambda i: (i, 0)),
            pl.BlockSpec((B, C16), lambda i: (i, 0)),
            pl.BlockSpec((D, H), full),
            pl.BlockSpec((H, H), full),
            pl.BlockSpec((1, H), full),
            pl.BlockSpec((H, D), full),
            pl.BlockSpec((1, D), full),
        ],
        out_specs=[
            pl.BlockSpec((B, D), lambda i: (i, 0)),
            pl.BlockSpec((B, C16), lambda i: (i, 0)),
        ],
        out_shape=[
            jax.ShapeDtypeStruct((N, D), jnp.float32),
            jax.ShapeDtypeStruct((N, C16), jnp.float32),
        ],
    )(nf, cpad, a0, a1, q0, q1, wn1h, wn1a, bn1, wn2, bn2)


def kernel(node_feats, coords, edge_index, edge_feats, W_e1, b_e1, W_e2,
           b_e2, W_n1, b_n1, W_n2, b_n2, W_c1, b_c1, W_c2, W_a, b_a):
    w1d = W_e1[0:D]
    w1s = W_e1[D:2 * D]
    wr = W_e1[2 * D:2 * D + 1]
    w1e = W_e1[2 * D + 1:]
    be1 = b_e1.reshape(1, H)

    td, ts = _k1(node_feats, w1d, w1s, be1)

    cpad = jnp.pad(coords, ((0, 0), (0, C16 - 3)))
    dst3 = edge_index[1].reshape(NW, NCH, CH)
    src3 = edge_index[0].reshape(NW, NCH, CH)

    k2w, k2n, k4w, k4n = _sc_kernels()
    gd, gs = k2w(td, ts, dst3, src3)
    cd, cs = k2n(cpad, dst3, src3)

    u, ucol = _k3(
        gd, gs, cd, cs,
        edge_feats.reshape(NW, NCH, CH, DE), w1e, wr, W_e2,
        b_e2.reshape(1, H), W_a, b_a.reshape(1, 1), W_c1,
        b_c1.reshape(1, H), W_c2)

    z128 = jnp.zeros((NP, H), jnp.float32)
    z16 = jnp.zeros((N, C16), jnp.float32)
    (o128,) = k4w(u, dst3, z128)
    (o16,) = k4n(ucol, dst3, z16)

    h_out, c_out = _k5(
        node_feats, cpad, o128[0], o128[1], o16[0], o16[1],
        W_n1[0:D], W_n1[D:], b_n1.reshape(1, H), W_n2, b_n2.reshape(1, D))

    return (h_out, c_out[:, 0:3])
